# speedup vs baseline: 36.4495x; 36.4495x over previous
import functools

import jax
import jax.numpy as jnp
from jax.experimental import pallas as pl
from jax.experimental.pallas import tpu as pltpu

_N = 4096
_D = 128
_K = 32
_ALPHA = 3.0
_R = 256
_ONE_BITS = 0x3F800000


def _emb_body(w1_ref, w2_ref, m1_ref, m2_ref):
    m1_ref[...] = jnp.tanh(_ALPHA * w1_ref[...])
    m2_ref[...] = jnp.tanh(_ALPHA * w2_ref[...])


def _block_body(m1b_ref, m2b_ref, m1_ref, m2_ref, out_ref, thr_ref):
    x1 = m1b_ref[...]
    x2 = m2b_ref[...]
    dims = (((1,), (1,)), ((), ()))
    s = jax.lax.dot_general(
        x1, m2_ref[...], dims, preferred_element_type=jnp.float32
    ) - jax.lax.dot_general(
        x2, m1_ref[...], dims, preferred_element_type=jnp.float32
    )
    a = jnp.maximum(jnp.tanh(_ALPHA * s), 0.0)
    bits = jax.lax.bitcast_convert_type(a + 0.0, jnp.int32)

    ones = jnp.where(bits == _ONE_BITS, 1, 0)
    c1 = jnp.sum(ones, axis=1, keepdims=True)

    thr_ref[...] = jnp.full((_R, 1), _ONE_BITS, jnp.int32)

    @pl.when(jnp.min(c1) < _K)
    def _slow_path():
        def body(_, carry):
            lo, hi = carry
            mid = (lo + hi) >> 1
            cnt = jnp.sum(jnp.where(bits >= mid, 1, 0), axis=1,
                          keepdims=True)
            ok = cnt >= _K
            return jnp.where(ok, mid, lo), jnp.where(ok, hi, mid)

        lo0 = jnp.zeros((_R, 1), jnp.int32)
        hi0 = jnp.full((_R, 1), _ONE_BITS + 1, jnp.int32)
        lo, _ = jax.lax.fori_loop(0, 31, body, (lo0, hi0))
        thr_ref[...] = lo

    thr = thr_ref[...]
    gt = bits > thr
    eq = bits == thr
    need = _K - jnp.sum(jnp.where(gt, 1, 0), axis=1, keepdims=True)
    e = jnp.where(eq, 1, 0)
    x = e
    sh = 1
    while sh < _N:
        x = x + jnp.concatenate(
            [jnp.zeros((_R, sh), jnp.int32), x[:, : _N - sh]], axis=1
        )
        sh *= 2
    keep_tie = eq & ((x - e) < need)
    mask = gt | keep_tie
    out_ref[...] = jnp.where(mask, a, 0.0)


@jax.jit
def kernel(W1, W2):
    m1, m2 = pl.pallas_call(
        _emb_body,
        out_shape=[
            jax.ShapeDtypeStruct((_N, _D), jnp.float32),
            jax.ShapeDtypeStruct((_N, _D), jnp.float32),
        ],
    )(W1, W2)

    grid = (_N // _R,)
    out = pl.pallas_call(
        _block_body,
        grid=grid,
        in_specs=[
            pl.BlockSpec((_R, _D), lambda i: (i, 0)),
            pl.BlockSpec((_R, _D), lambda i: (i, 0)),
            pl.BlockSpec((_N, _D), lambda i: (0, 0)),
            pl.BlockSpec((_N, _D), lambda i: (0, 0)),
        ],
        out_specs=pl.BlockSpec((_R, _N), lambda i: (i, 0)),
        out_shape=jax.ShapeDtypeStruct((_N, _N), jnp.float32),
        scratch_shapes=[pltpu.VMEM((_R, 1), jnp.int32)],
    )(m1, m2, m1, m2)
    return out


# speedup vs baseline: 190.8351x; 5.2356x over previous
import functools

import jax
import jax.numpy as jnp
from jax.experimental import pallas as pl
from jax.experimental.pallas import tpu as pltpu

_N = 4096
_D = 128
_K = 32
_ALPHA = 3.0
_R = 256
_ONE_BITS = 0x3F800000


def _emb_body(w1_ref, w2_ref, m1_ref, m2_ref):
    m1_ref[...] = jnp.tanh(_ALPHA * w1_ref[...])
    m2_ref[...] = jnp.tanh(_ALPHA * w2_ref[...])


_W = 256


def _block_body(m1b_ref, m2b_ref, m1_ref, m2_ref, out_ref, thr_ref):
    x1 = m1b_ref[...]
    x2 = m2b_ref[...]
    dims = (((1,), (1,)), ((), ()))

    sw = jax.lax.dot_general(
        x1, m2_ref[0:_W, :], dims, preferred_element_type=jnp.float32
    ) - jax.lax.dot_general(
        x2, m1_ref[0:_W, :], dims, preferred_element_type=jnp.float32
    )
    aw = jnp.maximum(jnp.tanh(_ALPHA * sw), 0.0)
    ew = jnp.where(aw == 1.0, 1.0, 0.0)
    cw = jnp.sum(ew, axis=1, keepdims=True)
    fast = jnp.min(cw) >= _K

    @pl.when(fast)
    def _fast_path():
        x = ew
        sh = 1
        while sh < _W:
            x = x + jnp.concatenate(
                [jnp.zeros((_R, sh), jnp.float32), x[:, : _W - sh]],
                axis=1,
            )
            sh *= 2
        keep = (ew > 0.0) & ((x - ew) < _K)
        out_ref[:, 0:_W] = jnp.where(keep, aw, 0.0)
        out_ref[:, _W:_N] = jnp.zeros((_R, _N - _W), jnp.float32)

    @pl.when(jnp.logical_not(fast))
    def _general_path():
        s = jax.lax.dot_general(
            x1, m2_ref[...], dims, preferred_element_type=jnp.float32
        ) - jax.lax.dot_general(
            x2, m1_ref[...], dims, preferred_element_type=jnp.float32
        )
        a = jnp.maximum(jnp.tanh(_ALPHA * s), 0.0)
        bits = jax.lax.bitcast_convert_type(a + 0.0, jnp.int32)

        ones = jnp.where(bits == _ONE_BITS, 1, 0)
        c1 = jnp.sum(ones, axis=1, keepdims=True)

        thr_ref[...] = jnp.full((_R, 1), _ONE_BITS, jnp.int32)

        @pl.when(jnp.min(c1) < _K)
        def _slow_path():
            def body(_, carry):
                lo, hi = carry
                mid = (lo + hi) >> 1
                cnt = jnp.sum(jnp.where(bits >= mid, 1, 0), axis=1,
                              keepdims=True)
                ok = cnt >= _K
                return jnp.where(ok, mid, lo), jnp.where(ok, hi, mid)

            lo0 = jnp.zeros((_R, 1), jnp.int32)
            hi0 = jnp.full((_R, 1), _ONE_BITS + 1, jnp.int32)
            lo, _ = jax.lax.fori_loop(0, 31, body, (lo0, hi0))
            thr_ref[...] = lo

        thr = thr_ref[...]
        gt = bits > thr
        eq = bits == thr
        need = _K - jnp.sum(jnp.where(gt, 1, 0), axis=1, keepdims=True)
        e = jnp.where(eq, 1, 0)
        x = e
        sh = 1
        while sh < _N:
            x = x + jnp.concatenate(
                [jnp.zeros((_R, sh), jnp.int32), x[:, : _N - sh]], axis=1
            )
            sh *= 2
        keep_tie = eq & ((x - e) < need)
        mask = gt | keep_tie
        out_ref[...] = jnp.where(mask, a, 0.0)


@jax.jit
def kernel(W1, W2):
    m1, m2 = pl.pallas_call(
        _emb_body,
        out_shape=[
            jax.ShapeDtypeStruct((_N, _D), jnp.float32),
            jax.ShapeDtypeStruct((_N, _D), jnp.float32),
        ],
    )(W1, W2)

    grid = (_N // _R,)
    out = pl.pallas_call(
        _block_body,
        grid=grid,
        in_specs=[
            pl.BlockSpec((_R, _D), lambda i: (i, 0)),
            pl.BlockSpec((_R, _D), lambda i: (i, 0)),
            pl.BlockSpec((_N, _D), lambda i: (0, 0)),
            pl.BlockSpec((_N, _D), lambda i: (0, 0)),
        ],
        out_specs=pl.BlockSpec((_R, _N), lambda i: (i, 0)),
        out_shape=jax.ShapeDtypeStruct((_N, _N), jnp.float32),
        scratch_shapes=[pltpu.VMEM((_R, 1), jnp.int32)],
    )(m1, m2, m1, m2)
    return out


# speedup vs baseline: 217.0790x; 1.1375x over previous
import functools

import jax
import jax.numpy as jnp
from jax.experimental import pallas as pl
from jax.experimental.pallas import tpu as pltpu

_N = 4096
_D = 128
_K = 32
_ALPHA = 3.0
_R = 512
_ONE_BITS = 0x3F800000


def _emb_body(w1_ref, w2_ref, m1_ref, m2_ref):
    m1_ref[...] = jnp.tanh(_ALPHA * w1_ref[...])
    m2_ref[...] = jnp.tanh(_ALPHA * w2_ref[...])


_W = 256


def _block_body(m1b_ref, m2b_ref, m1_ref, m2_ref, out_ref, thr_ref):
    x1 = m1b_ref[...]
    x2 = m2b_ref[...]
    dims = (((1,), (1,)), ((), ()))

    sw = jax.lax.dot_general(
        x1, m2_ref[0:_W, :], dims, preferred_element_type=jnp.float32
    ) - jax.lax.dot_general(
        x2, m1_ref[0:_W, :], dims, preferred_element_type=jnp.float32
    )
    aw = jnp.maximum(jnp.tanh(_ALPHA * sw), 0.0)
    ew = jnp.where(aw == 1.0, 1.0, 0.0)
    cw = jnp.sum(ew, axis=1, keepdims=True)
    fast = jnp.min(cw) >= _K

    @pl.when(fast)
    def _fast_path():
        x = ew
        sh = 1
        while sh < _W:
            x = x + jnp.concatenate(
                [jnp.zeros((_R, sh), jnp.float32), x[:, : _W - sh]],
                axis=1,
            )
            sh *= 2
        keep = (ew > 0.0) & ((x - ew) < _K)
        out_ref[:, 0:_W] = jnp.where(keep, aw, 0.0)
        out_ref[:, _W:_N] = jnp.zeros((_R, _N - _W), jnp.float32)

    @pl.when(jnp.logical_not(fast))
    def _general_path():
        s = jax.lax.dot_general(
            x1, m2_ref[...], dims, preferred_element_type=jnp.float32
        ) - jax.lax.dot_general(
            x2, m1_ref[...], dims, preferred_element_type=jnp.float32
        )
        a = jnp.maximum(jnp.tanh(_ALPHA * s), 0.0)
        bits = jax.lax.bitcast_convert_type(a + 0.0, jnp.int32)

        ones = jnp.where(bits == _ONE_BITS, 1, 0)
        c1 = jnp.sum(ones, axis=1, keepdims=True)

        thr_ref[...] = jnp.full((_R, 1), _ONE_BITS, jnp.int32)

        @pl.when(jnp.min(c1) < _K)
        def _slow_path():
            def body(_, carry):
                lo, hi = carry
                mid = (lo + hi) >> 1
                cnt = jnp.sum(jnp.where(bits >= mid, 1, 0), axis=1,
                              keepdims=True)
                ok = cnt >= _K
                return jnp.where(ok, mid, lo), jnp.where(ok, hi, mid)

            lo0 = jnp.zeros((_R, 1), jnp.int32)
            hi0 = jnp.full((_R, 1), _ONE_BITS + 1, jnp.int32)
            lo, _ = jax.lax.fori_loop(0, 31, body, (lo0, hi0))
            thr_ref[...] = lo

        thr = thr_ref[...]
        gt = bits > thr
        eq = bits == thr
        need = _K - jnp.sum(jnp.where(gt, 1, 0), axis=1, keepdims=True)
        e = jnp.where(eq, 1, 0)
        x = e
        sh = 1
        while sh < _N:
            x = x + jnp.concatenate(
                [jnp.zeros((_R, sh), jnp.int32), x[:, : _N - sh]], axis=1
            )
            sh *= 2
        keep_tie = eq & ((x - e) < need)
        mask = gt | keep_tie
        out_ref[...] = jnp.where(mask, a, 0.0)


@jax.jit
def kernel(W1, W2):
    m1, m2 = pl.pallas_call(
        _emb_body,
        out_shape=[
            jax.ShapeDtypeStruct((_N, _D), jnp.float32),
            jax.ShapeDtypeStruct((_N, _D), jnp.float32),
        ],
    )(W1, W2)

    grid = (_N // _R,)
    out = pl.pallas_call(
        _block_body,
        grid=grid,
        in_specs=[
            pl.BlockSpec((_R, _D), lambda i: (i, 0)),
            pl.BlockSpec((_R, _D), lambda i: (i, 0)),
            pl.BlockSpec((_N, _D), lambda i: (0, 0)),
            pl.BlockSpec((_N, _D), lambda i: (0, 0)),
        ],
        out_specs=pl.BlockSpec((_R, _N), lambda i: (i, 0)),
        out_shape=jax.ShapeDtypeStruct((_N, _N), jnp.float32),
        scratch_shapes=[pltpu.VMEM((_R, 1), jnp.int32)],
    )(m1, m2, m1, m2)
    return out
